# baseline (speedup 1.0000x reference)
import jax
import jax.numpy as jnp
from jax.experimental import pallas as pl

_ROWS = 400


def _lin1_body(x_ref, w1t_ref, o_ref):
    xp = jnp.dot(x_ref[...], w1t_ref[...], preferred_element_type=jnp.float32)
    o_ref[...] = xp.astype(jnp.bfloat16)


def _prop1_body(c_ref, xp_ref, b1_ref, w2t_ref, o_ref):
    c = c_ref[...].astype(jnp.bfloat16)
    h = jax.lax.dot_general(
        c, xp_ref[...], (((1,), (0,)), ((), ())),
        preferred_element_type=jnp.float32)
    h = jnp.maximum(h + b1_ref[...], 0.0)
    o_ref[...] = jnp.dot(h.astype(jnp.bfloat16), w2t_ref[...],
                         preferred_element_type=jnp.float32).astype(jnp.bfloat16)


def _prop2_body(c_ref, hp_ref, b2_ref, o_ref):
    c = c_ref[...].astype(jnp.bfloat16)
    o_ref[...] = jax.lax.dot_general(
        c, hp_ref[...], (((1,), (0,)), ((), ())),
        preferred_element_type=jnp.float32) + b2_ref[...]


def kernel(x, edge_index, C, W1, b1, W2, b2):
    del edge_index
    n, in_dim = x.shape
    hid = W1.shape[0]
    ncls = W2.shape[0]
    blocks = n // _ROWS

    xp = pl.pallas_call(
        _lin1_body,
        out_shape=jax.ShapeDtypeStruct((n, hid), jnp.bfloat16),
    )(x, W1.T)

    h1p = pl.pallas_call(
        _prop1_body,
        grid=(blocks,),
        in_specs=[
            pl.BlockSpec((_ROWS, n), lambda i: (i, 0)),
            pl.BlockSpec((n, hid), lambda i: (0, 0)),
            pl.BlockSpec((1, hid), lambda i: (0, 0)),
            pl.BlockSpec((hid, ncls), lambda i: (0, 0)),
        ],
        out_specs=pl.BlockSpec((_ROWS, ncls), lambda i: (i, 0)),
        out_shape=jax.ShapeDtypeStruct((n, ncls), jnp.bfloat16),
    )(C, xp, b1.reshape(1, hid), W2.T.astype(jnp.bfloat16))

    out = pl.pallas_call(
        _prop2_body,
        grid=(blocks,),
        in_specs=[
            pl.BlockSpec((_ROWS, n), lambda i: (i, 0)),
            pl.BlockSpec((n, ncls), lambda i: (0, 0)),
            pl.BlockSpec((1, ncls), lambda i: (0, 0)),
        ],
        out_specs=pl.BlockSpec((_ROWS, ncls), lambda i: (i, 0)),
        out_shape=jax.ShapeDtypeStruct((n, ncls), jnp.float32),
    )(C, h1p, b2.reshape(1, ncls))

    return out


# speedup vs baseline: 1.0184x; 1.0184x over previous
import jax
import jax.numpy as jnp
from jax.experimental import pallas as pl
from jax.experimental.pallas import tpu as pltpu

_ROWS = 400


def _fused_body(x_ref, w1t_ref, b1_ref, w2t_ref, b2_ref, c_ref, o_ref,
                xp_ref, hp_ref):
    i = pl.program_id(0)
    nb = pl.num_programs(0) // 2

    @pl.when(i == 0)
    def _():
        xp = jnp.dot(x_ref[...], w1t_ref[...],
                     preferred_element_type=jnp.float32)
        xp_ref[...] = xp.astype(jnp.bfloat16)

    c = c_ref[...].astype(jnp.bfloat16)

    @pl.when(i < nb)
    def _():
        h = jax.lax.dot_general(
            c, xp_ref[...], (((1,), (0,)), ((), ())),
            preferred_element_type=jnp.float32)
        h = jnp.maximum(h + b1_ref[...], 0.0)
        hp = jnp.dot(h.astype(jnp.bfloat16), w2t_ref[...],
                     preferred_element_type=jnp.float32)
        hp_ref[pl.ds(i * _ROWS, _ROWS), :] = hp.astype(jnp.bfloat16)

    @pl.when(i >= nb)
    def _():
        o_ref[...] = jax.lax.dot_general(
            c, hp_ref[...], (((1,), (0,)), ((), ())),
            preferred_element_type=jnp.float32) + b2_ref[...]


def kernel(x, edge_index, C, W1, b1, W2, b2):
    del edge_index
    n, in_dim = x.shape
    hid = W1.shape[0]
    ncls = W2.shape[0]
    nb = n // _ROWS

    return pl.pallas_call(
        _fused_body,
        grid=(2 * nb,),
        in_specs=[
            pl.BlockSpec((n, in_dim), lambda i: (0, 0)),
            pl.BlockSpec((in_dim, hid), lambda i: (0, 0)),
            pl.BlockSpec((1, hid), lambda i: (0, 0)),
            pl.BlockSpec((hid, ncls), lambda i: (0, 0)),
            pl.BlockSpec((1, ncls), lambda i: (0, 0)),
            pl.BlockSpec((_ROWS, n), lambda i: (i % (pl.num_programs(0) // 2), 0)),
        ],
        out_specs=pl.BlockSpec((_ROWS, ncls), lambda i: (i % (pl.num_programs(0) // 2), 0)),
        out_shape=jax.ShapeDtypeStruct((n, ncls), jnp.float32),
        scratch_shapes=[
            pltpu.VMEM((n, hid), jnp.bfloat16),
            pltpu.VMEM((n, ncls), jnp.bfloat16),
        ],
    )(x, W1.T, b1.reshape(1, hid), W2.T.astype(jnp.bfloat16),
      b2.reshape(1, ncls), C)
